# baseline (speedup 1.0000x reference)
import functools

import jax
import jax.numpy as jnp
from jax import lax
from jax.experimental import pallas as pl
from jax.experimental.pallas import tpu as pltpu
from jax.experimental.pallas import tpu_sc as plsc

N = 10000
E = 320000
D = 128
HW = 64
LAT = 64
G = 16
MN = 1024
NC, NS = 2, 16

IROWS = E // 128
IPAD = 2560
IRX = IPAD + 8
NTRASH = 512
DRPW = IPAD // (NC * NS)
DCH = 8
ARPT = IPAD // NS
ACH = 4
PCH = ARPT // ACH
CPT = 624
TAIL = N - CPT * NS
CC = 104

_SC_PARAMS = pltpu.CompilerParams(use_tc_tiling_on_sc=False)


def _sc_mesh():
    return plsc.VectorSubcoreMesh(core_axis_name="c", subcore_axis_name="s")


def _deg_body(dst_hbm, ones_hbm, zeros_hbm, out_hbm, dbuf, ones_v, stage, acc,
              sem):
    c = lax.axis_index("c")
    s = lax.axis_index("s")
    pltpu.sync_copy(zeros_hbm, stage)
    pltpu.sync_copy(stage.at[pl.ds(0, CPT)], acc.at[pl.ds(s * CPT, CPT)])

    @pl.when(s == NS - 1)
    def _():
        pltpu.sync_copy(stage.at[pl.ds(0, TAIL)],
                        acc.at[pl.ds(CPT * NS, TAIL)])

    pltpu.sync_copy(ones_hbm, ones_v)
    base = (c * NS + s) * DRPW
    pltpu.sync_copy(dst_hbm.at[pl.ds(base, DRPW), :], dbuf)
    plsc.subcore_barrier()

    def chunk(i, carry):
        descs = [
            pltpu.async_copy(ones_v, acc.at[dbuf.at[i * DCH + j]], sem,
                             add=True)
            for j in range(DCH)
        ]
        for d in descs:
            d.wait()
        return carry

    lax.fori_loop(0, DRPW // DCH, chunk, 0)
    plsc.subcore_barrier()
    pltpu.sync_copy(acc.at[pl.ds(s * CPT, CPT)], stage.at[pl.ds(0, CPT)])
    pltpu.sync_copy(stage.at[pl.ds(0, CPT)],
                    out_hbm.at[pl.ds(c * N + s * CPT, CPT)])

    @pl.when(s == NS - 1)
    def _():
        pltpu.sync_copy(acc.at[pl.ds(CPT * NS, TAIL)],
                        stage.at[pl.ds(CPT, TAIL)])
        pltpu.sync_copy(stage.at[pl.ds(CPT, TAIL)],
                        out_hbm.at[pl.ds(c * N + CPT * NS, TAIL)])


def _sc_degree(dst3, ones128, zeros1):
    return pl.kernel(
        _deg_body,
        out_type=jax.ShapeDtypeStruct((NC * N,), jnp.float32),
        mesh=_sc_mesh(),
        scratch_types=[
            pltpu.VMEM((DRPW, 128), jnp.int32),
            pltpu.VMEM((128,), jnp.float32),
            pltpu.VMEM((CPT + TAIL,), jnp.float32),
            pltpu.VMEM_SHARED((N + NTRASH,), jnp.float32),
            pltpu.SemaphoreType.DMA,
        ],
        compiler_params=_SC_PARAMS,
    )(dst3, ones128, zeros1)


def _agg_body(hs_hbm, srcs_hbm, dst_hbm, zeros_hbm, out_hbm,
              sbufA, dbufA, rows, stage, acc, semA):
    c = lax.axis_index("c")
    s = lax.axis_index("s")
    pltpu.sync_copy(zeros_hbm, stage)
    for k in range(CPT // CC):
        pltpu.sync_copy(stage, acc.at[pl.ds(s * CPT + k * CC, CC)])

    @pl.when(s == NS - 1)
    def _():
        pltpu.sync_copy(stage.at[pl.ds(0, TAIL), :],
                        acc.at[pl.ds(CPT * NS, TAIL)])

    base = s * ARPT
    plsc.subcore_barrier()

    def chunk(i, carry):
        rb = base + i * ACH
        pltpu.sync_copy(srcs_hbm.at[c, pl.ds(rb, ACH), :], sbufA)
        pltpu.sync_copy(dst_hbm.at[pl.ds(rb, ACH), :], dbufA)
        descs = [
            pltpu.async_copy(hs_hbm.at[sbufA.at[j]],
                             rows.at[pl.ds(j * 128, 128)], semA)
            for j in range(ACH)
        ]
        for d in descs:
            d.wait()
        for j in range(ACH):
            pltpu.sync_copy(rows.at[pl.ds(j * 128, 128)],
                            acc.at[dbufA.at[j]], add=True)
        return carry

    lax.fori_loop(0, PCH, chunk, 0)
    plsc.subcore_barrier()
    for k in range(CPT // CC):
        pltpu.sync_copy(acc.at[pl.ds(s * CPT + k * CC, CC)], stage)
        pltpu.sync_copy(stage, out_hbm.at[c, pl.ds(s * CPT + k * CC, CC), :])

    @pl.when(s == NS - 1)
    def _():
        pltpu.sync_copy(acc.at[pl.ds(CPT * NS, TAIL)],
                        stage.at[pl.ds(0, TAIL), :])
        pltpu.sync_copy(stage.at[pl.ds(0, TAIL), :],
                        out_hbm.at[c, pl.ds(CPT * NS, TAIL), :])


def _sc_aggregate(hs_flat, srcs3, dst3, zerosC):
    return pl.kernel(
        _agg_body,
        out_type=jax.ShapeDtypeStruct((NC, N, HW), jnp.float32),
        mesh=_sc_mesh(),
        scratch_types=[
            pltpu.VMEM((ACH, 128), jnp.int32),
            pltpu.VMEM((ACH, 128), jnp.int32),
            pltpu.VMEM((ACH * 128, HW), jnp.float32),
            pltpu.VMEM((CC, HW), jnp.float32),
            pltpu.VMEM_SHARED((N + NTRASH, HW), jnp.float32),
            pltpu.SemaphoreType.DMA,
        ],
        compiler_params=_SC_PARAMS,
    )(hs_flat, srcs3, dst3, zerosC)


def _mm1_body(x_ref, w_ref, h_ref):
    h_ref[0] = jnp.dot(x_ref[...], w_ref[0], preferred_element_type=jnp.float32,
                       precision=lax.Precision.HIGHEST)


def _tc_mm1(x, w1s):
    return pl.pallas_call(
        _mm1_body,
        grid=(2,),
        in_specs=[
            pl.BlockSpec((N, D), lambda f: (0, 0)),
            pl.BlockSpec((1, D, HW), lambda f: (f, 0, 0)),
        ],
        out_specs=pl.BlockSpec((1, N, HW), lambda f: (f, 0, 0)),
        out_shape=jax.ShapeDtypeStruct((NC, N, HW), jnp.float32),
    )(x, w1s)


def _scale1_body(deg_ref, h_ref, hs_ref, dinv_ref):
    degsum = deg_ref[0] + deg_ref[1] + 1.0
    dinv = lax.rsqrt(degsum)
    dinv_ref[...] = dinv
    hs_ref[0] = h_ref[0] * dinv
    hs_ref[1] = h_ref[1] * dinv


def _tc_scale1(deg2, h1):
    return pl.pallas_call(
        _scale1_body,
        out_shape=[
            jax.ShapeDtypeStruct((NC, N, HW), jnp.float32),
            jax.ShapeDtypeStruct((N, 1), jnp.float32),
        ],
    )(deg2, h1)


def _tc2_body(agg_ref, hs_ref, dinv_ref, b_ref, w_ref, out_ref):
    dinv = dinv_ref[...]
    h0 = dinv * (agg_ref[0] + hs_ref[0])
    h1 = dinv * (agg_ref[1] + hs_ref[1])
    h = jnp.concatenate([h0, h1], axis=1) + b_ref[...]
    h = jnp.maximum(h, 0.0)
    h2 = jnp.dot(h, w_ref[...], preferred_element_type=jnp.float32,
                 precision=lax.Precision.HIGHEST)
    out_ref[0] = dinv * h2[:, :HW]
    out_ref[1] = dinv * h2[:, HW:]


NB = 2000


def _tc2(agg, hs, dinv, b1, w2):
    return pl.pallas_call(
        _tc2_body,
        grid=(N // NB,),
        in_specs=[
            pl.BlockSpec((NC, NB, HW), lambda i: (0, i, 0)),
            pl.BlockSpec((NC, NB, HW), lambda i: (0, i, 0)),
            pl.BlockSpec((NB, 1), lambda i: (i, 0)),
            pl.BlockSpec((1, D), lambda i: (0, 0)),
            pl.BlockSpec((D, D), lambda i: (0, 0)),
        ],
        out_specs=pl.BlockSpec((NC, NB, HW), lambda i: (0, i, 0)),
        out_shape=jax.ShapeDtypeStruct((NC, N, HW), jnp.float32),
    )(agg, hs, dinv, b1, w2)


ZP = N + NB


def _tc3_body(agg_ref, hs_ref, dinv_ref, b_ref, wmu_ref, bmu_ref, wlv_ref,
              blv_ref, eps_ref, batch_ref,
              mu_ref, lv_ref, zpad_ref, offs_ref, cnts_ref):
    i = pl.program_id(0)
    dinv = dinv_ref[...]
    h0 = dinv * (agg_ref[0] + hs_ref[0])
    h1 = dinv * (agg_ref[1] + hs_ref[1])
    h = jnp.concatenate([h0, h1], axis=1) + b_ref[...]
    h = jnp.maximum(h, 0.0)
    mu = jnp.dot(h, wmu_ref[...], preferred_element_type=jnp.float32,
                 precision=lax.Precision.HIGHEST) + bmu_ref[...]
    logvar = jnp.dot(h, wlv_ref[...], preferred_element_type=jnp.float32,
                     precision=lax.Precision.HIGHEST) + blv_ref[...]
    mu_ref[...] = mu
    lv_ref[...] = logvar
    lvc = jnp.clip(logvar, -20.0, 20.0)
    z = mu + eps_ref[...] * jnp.exp(0.5 * lvc)
    zpad_ref[...] = jnp.where(i < N // NB, z, 0.0)
    gids = lax.broadcasted_iota(jnp.int32, (G, N), 0)
    b = batch_ref[...]
    cnts_ref[...] = jnp.sum((b == gids).astype(jnp.int32), axis=1).reshape(1, G)
    offs_ref[...] = jnp.sum((b < gids).astype(jnp.int32), axis=1).reshape(1, G)


def _tc3(agg, hs, dinv, b2, wmu, bmu, wlv, blv, eps, batch2):
    clamp = lambda i: jnp.minimum(i, N // NB - 1)
    return pl.pallas_call(
        _tc3_body,
        grid=(ZP // NB,),
        in_specs=[
            pl.BlockSpec((NC, NB, HW), lambda i: (0, clamp(i), 0)),
            pl.BlockSpec((NC, NB, HW), lambda i: (0, clamp(i), 0)),
            pl.BlockSpec((NB, 1), lambda i: (clamp(i), 0)),
            pl.BlockSpec((1, D), lambda i: (0, 0)),
            pl.BlockSpec((D, LAT), lambda i: (0, 0)),
            pl.BlockSpec((1, LAT), lambda i: (0, 0)),
            pl.BlockSpec((D, LAT), lambda i: (0, 0)),
            pl.BlockSpec((1, LAT), lambda i: (0, 0)),
            pl.BlockSpec((NB, LAT), lambda i: (clamp(i), 0)),
            pl.BlockSpec((1, N), lambda i: (0, 0)),
        ],
        out_specs=[
            pl.BlockSpec((NB, LAT), lambda i: (clamp(i), 0)),
            pl.BlockSpec((NB, LAT), lambda i: (clamp(i), 0)),
            pl.BlockSpec((NB, LAT), lambda i: (i, 0)),
            pl.BlockSpec((1, G), lambda i: (0, 0)),
            pl.BlockSpec((1, G), lambda i: (0, 0)),
        ],
        out_shape=[
            jax.ShapeDtypeStruct((N, LAT), jnp.float32),
            jax.ShapeDtypeStruct((N, LAT), jnp.float32),
            jax.ShapeDtypeStruct((ZP, LAT), jnp.float32),
            jax.ShapeDtypeStruct((1, G), jnp.int32),
            jax.ShapeDtypeStruct((1, G), jnp.int32),
        ],
    )(agg, hs, dinv, b2, wmu, bmu, wlv, blv, eps, batch2)


def _dec_body(offs_ref, cnts_ref, z_ref, bias_ref, adj_ref, mask_ref):
    g = pl.program_id(0)
    off = offs_ref[g]
    cnt = cnts_ref[g]
    zb = z_ref[pl.ds(off, MN), :]
    colmask = lax.broadcasted_iota(jnp.int32, (MN, 1), 0) < cnt
    zm = jnp.where(colmask, zb, 0.0)
    logits = lax.dot_general(zm, zm, (((1,), (1,)), ((), ())),
                             preferred_element_type=jnp.float32,
                             precision=lax.Precision.HIGHEST)
    logits = logits * (LAT ** -0.5) + bias_ref[0]
    adj_ref[0] = jax.nn.sigmoid(logits)
    rowmask = lax.broadcasted_iota(jnp.int32, (1, MN), 1) < cnt
    mask_ref[0] = rowmask.astype(jnp.int32)


def _tc_decoder(offs, cnts, zpad, bias1):
    grid_spec = pltpu.PrefetchScalarGridSpec(
        num_scalar_prefetch=2,
        grid=(G,),
        in_specs=[
            pl.BlockSpec((ZP, LAT), lambda g, o, c: (0, 0)),
            pl.BlockSpec((1,), lambda g, o, c: (0,)),
        ],
        out_specs=[
            pl.BlockSpec((1, MN, MN), lambda g, o, c: (g, 0, 0)),
            pl.BlockSpec((1, 1, MN), lambda g, o, c: (g, 0, 0)),
        ],
    )
    return pl.pallas_call(
        _dec_body,
        grid_spec=grid_spec,
        out_shape=[
            jax.ShapeDtypeStruct((G, MN, MN), jnp.float32),
            jax.ShapeDtypeStruct((G, 1, MN), jnp.int32),
        ],
    )(offs, cnts, zpad, bias1)


def kernel(x, edge_index, batch, eps, W1, b1, W2, b2, Wmu, bmu, Wlv, blv,
           dec_bias):
    src = edge_index[0]
    dst = edge_index[1]
    npad = IRX * 128 - E
    srcp = jnp.concatenate([src, jnp.zeros((npad,), jnp.int32)])
    trash = N + jnp.arange(npad, dtype=jnp.int32) % NTRASH
    dstp = jnp.concatenate([dst, trash])
    dst3 = dstp.reshape(IRX, 128)
    srcs3 = jnp.stack([srcp, srcp + N], axis=0).reshape(NC, IRX, 128)
    ones128 = jnp.ones((128,), jnp.float32)
    zeros1 = jnp.zeros((CPT + TAIL,), jnp.float32)
    zerosC = jnp.zeros((CC, HW), jnp.float32)

    deg2 = _sc_degree(dst3, ones128, zeros1)

    w1s = jnp.stack([W1[:, :HW], W1[:, HW:]], axis=0)
    h1 = _tc_mm1(x + 0.0 * deg2[:N].reshape(N, 1), w1s)
    hs1, dinv = _tc_scale1(deg2.reshape(NC, N, 1), h1)
    agg1 = _sc_aggregate(hs1.reshape(NC * N, HW), srcs3, dst3, zerosC)

    hs2 = _tc2(agg1, hs1, dinv, b1.reshape(1, D), W2)
    agg2 = _sc_aggregate(hs2.reshape(NC * N, HW), srcs3, dst3, zerosC)

    mu, logvar, zpad, offs, cnts = _tc3(
        agg2, hs2, dinv, b2.reshape(1, D), Wmu, bmu.reshape(1, LAT),
        Wlv, blv.reshape(1, LAT), eps, batch.reshape(1, N))

    adj, maski = _tc_decoder(offs.reshape(G), cnts.reshape(G), zpad,
                             dec_bias.reshape(1))
    return adj, mu, logvar, maski.reshape(G, MN).astype(bool)


# speedup vs baseline: 1.0007x; 1.0007x over previous
import functools

import jax
import jax.numpy as jnp
from jax import lax
from jax.experimental import pallas as pl
from jax.experimental.pallas import tpu as pltpu
from jax.experimental.pallas import tpu_sc as plsc

N = 10000
E = 320000
D = 128
HW = 64
LAT = 64
G = 16
MN = 1024
NC, NS = 2, 16

IROWS = E // 128
IPAD = 2560
IRX = IPAD + 8
NTRASH = 512
DRPW = IPAD // (NC * NS)
DCH = 8
ARPT = IPAD // NS
ACH = 4
PCH = ARPT // ACH
CPT = 624
TAIL = N - CPT * NS
CC = 104

_SC_PARAMS = pltpu.CompilerParams(use_tc_tiling_on_sc=False)


def _sc_mesh():
    return plsc.VectorSubcoreMesh(core_axis_name="c", subcore_axis_name="s")


def _deg_body(dst_hbm, ones_hbm, zeros_hbm, out_hbm, dbuf, ones_v, stage, acc,
              sem):
    c = lax.axis_index("c")
    s = lax.axis_index("s")
    pltpu.sync_copy(zeros_hbm, stage)
    pltpu.sync_copy(stage.at[pl.ds(0, CPT)], acc.at[pl.ds(s * CPT, CPT)])

    @pl.when(s == NS - 1)
    def _():
        pltpu.sync_copy(stage.at[pl.ds(0, TAIL)],
                        acc.at[pl.ds(CPT * NS, TAIL)])

    pltpu.sync_copy(ones_hbm, ones_v)
    base = (c * NS + s) * DRPW
    pltpu.sync_copy(dst_hbm.at[pl.ds(base, DRPW), :], dbuf)
    plsc.subcore_barrier()

    ndch = jnp.where(s >= 0, DRPW // DCH, 0)

    def chunk(i, carry):
        descs = [
            pltpu.async_copy(ones_v, acc.at[dbuf.at[i * DCH + j]], sem,
                             add=True)
            for j in range(DCH)
        ]
        for d in descs:
            d.wait()
        return carry

    lax.fori_loop(0, ndch, chunk, 0)
    plsc.subcore_barrier()
    pltpu.sync_copy(acc.at[pl.ds(s * CPT, CPT)], stage.at[pl.ds(0, CPT)])
    pltpu.sync_copy(stage.at[pl.ds(0, CPT)],
                    out_hbm.at[pl.ds(c * N + s * CPT, CPT)])

    @pl.when(s == NS - 1)
    def _():
        pltpu.sync_copy(acc.at[pl.ds(CPT * NS, TAIL)],
                        stage.at[pl.ds(CPT, TAIL)])
        pltpu.sync_copy(stage.at[pl.ds(CPT, TAIL)],
                        out_hbm.at[pl.ds(c * N + CPT * NS, TAIL)])


def _sc_degree(dst3, ones128, zeros1):
    return pl.kernel(
        _deg_body,
        out_type=jax.ShapeDtypeStruct((NC * N,), jnp.float32),
        mesh=_sc_mesh(),
        scratch_types=[
            pltpu.VMEM((DRPW, 128), jnp.int32),
            pltpu.VMEM((128,), jnp.float32),
            pltpu.VMEM((CPT + TAIL,), jnp.float32),
            pltpu.VMEM_SHARED((N + NTRASH,), jnp.float32),
            pltpu.SemaphoreType.DMA,
        ],
        compiler_params=_SC_PARAMS,
    )(dst3, ones128, zeros1)


def _agg_body(hs_hbm, srcs_hbm, dst_hbm, zeros_hbm, out_hbm,
              sbufA, dbufA, rows, stage, acc, semA):
    c = lax.axis_index("c")
    s = lax.axis_index("s")
    pltpu.sync_copy(zeros_hbm, stage)
    for k in range(CPT // CC):
        pltpu.sync_copy(stage, acc.at[pl.ds(s * CPT + k * CC, CC)])

    @pl.when(s == NS - 1)
    def _():
        pltpu.sync_copy(stage.at[pl.ds(0, TAIL), :],
                        acc.at[pl.ds(CPT * NS, TAIL)])

    base = s * ARPT
    plsc.subcore_barrier()
    nch = jnp.where(s >= 0, PCH, 0)

    def chunk(i, carry):
        rb = base + i * ACH
        pltpu.sync_copy(srcs_hbm.at[c, pl.ds(rb, ACH), :], sbufA)
        pltpu.sync_copy(dst_hbm.at[pl.ds(rb, ACH), :], dbufA)
        descs = [
            pltpu.async_copy(hs_hbm.at[sbufA.at[j]],
                             rows.at[pl.ds(j * 128, 128)], semA)
            for j in range(ACH)
        ]
        for d in descs:
            d.wait()
        for j in range(ACH):
            pltpu.sync_copy(rows.at[pl.ds(j * 128, 128)],
                            acc.at[dbufA.at[j]], add=True)
        return carry

    lax.fori_loop(0, nch, chunk, 0)
    plsc.subcore_barrier()
    for k in range(CPT // CC):
        pltpu.sync_copy(acc.at[pl.ds(s * CPT + k * CC, CC)], stage)
        pltpu.sync_copy(stage, out_hbm.at[c, pl.ds(s * CPT + k * CC, CC), :])

    @pl.when(s == NS - 1)
    def _():
        pltpu.sync_copy(acc.at[pl.ds(CPT * NS, TAIL)],
                        stage.at[pl.ds(0, TAIL), :])
        pltpu.sync_copy(stage.at[pl.ds(0, TAIL), :],
                        out_hbm.at[c, pl.ds(CPT * NS, TAIL), :])


def _sc_aggregate(hs_flat, srcs3, dst3, zerosC):
    return pl.kernel(
        _agg_body,
        out_type=jax.ShapeDtypeStruct((NC, N, HW), jnp.float32),
        mesh=_sc_mesh(),
        scratch_types=[
            pltpu.VMEM((ACH, 128), jnp.int32),
            pltpu.VMEM((ACH, 128), jnp.int32),
            pltpu.VMEM((ACH * 128, HW), jnp.float32),
            pltpu.VMEM((CC, HW), jnp.float32),
            pltpu.VMEM_SHARED((N + NTRASH, HW), jnp.float32),
            pltpu.SemaphoreType.DMA,
        ],
        compiler_params=_SC_PARAMS,
    )(hs_flat, srcs3, dst3, zerosC)


def _mm1_body(x_ref, w_ref, h_ref):
    h_ref[0] = jnp.dot(x_ref[...], w_ref[0], preferred_element_type=jnp.float32,
                       precision=lax.Precision.HIGHEST)


def _tc_mm1(x, w1s):
    return pl.pallas_call(
        _mm1_body,
        grid=(2,),
        in_specs=[
            pl.BlockSpec((N, D), lambda f: (0, 0)),
            pl.BlockSpec((1, D, HW), lambda f: (f, 0, 0)),
        ],
        out_specs=pl.BlockSpec((1, N, HW), lambda f: (f, 0, 0)),
        out_shape=jax.ShapeDtypeStruct((NC, N, HW), jnp.float32),
    )(x, w1s)


def _scale1_body(deg_ref, h_ref, hs_ref, dinv_ref):
    degsum = deg_ref[0] + deg_ref[1] + 1.0
    dinv = lax.rsqrt(degsum)
    dinv_ref[...] = dinv
    hs_ref[0] = h_ref[0] * dinv
    hs_ref[1] = h_ref[1] * dinv


def _tc_scale1(deg2, h1):
    return pl.pallas_call(
        _scale1_body,
        out_shape=[
            jax.ShapeDtypeStruct((NC, N, HW), jnp.float32),
            jax.ShapeDtypeStruct((N, 1), jnp.float32),
        ],
    )(deg2, h1)


def _tc2_body(agg_ref, hs_ref, dinv_ref, b_ref, w_ref, out_ref):
    dinv = dinv_ref[...]
    h0 = dinv * (agg_ref[0] + hs_ref[0])
    h1 = dinv * (agg_ref[1] + hs_ref[1])
    h = jnp.concatenate([h0, h1], axis=1) + b_ref[...]
    h = jnp.maximum(h, 0.0)
    h2 = jnp.dot(h, w_ref[...], preferred_element_type=jnp.float32,
                 precision=lax.Precision.HIGHEST)
    out_ref[0] = dinv * h2[:, :HW]
    out_ref[1] = dinv * h2[:, HW:]


NB = 2000


def _tc2(agg, hs, dinv, b1, w2):
    return pl.pallas_call(
        _tc2_body,
        grid=(N // NB,),
        in_specs=[
            pl.BlockSpec((NC, NB, HW), lambda i: (0, i, 0)),
            pl.BlockSpec((NC, NB, HW), lambda i: (0, i, 0)),
            pl.BlockSpec((NB, 1), lambda i: (i, 0)),
            pl.BlockSpec((1, D), lambda i: (0, 0)),
            pl.BlockSpec((D, D), lambda i: (0, 0)),
        ],
        out_specs=pl.BlockSpec((NC, NB, HW), lambda i: (0, i, 0)),
        out_shape=jax.ShapeDtypeStruct((NC, N, HW), jnp.float32),
    )(agg, hs, dinv, b1, w2)


ZP = N + NB


def _tc3_body(agg_ref, hs_ref, dinv_ref, b_ref, wmu_ref, bmu_ref, wlv_ref,
              blv_ref, eps_ref, batch_ref,
              mu_ref, lv_ref, zpad_ref, offs_ref, cnts_ref):
    i = pl.program_id(0)
    dinv = dinv_ref[...]
    h0 = dinv * (agg_ref[0] + hs_ref[0])
    h1 = dinv * (agg_ref[1] + hs_ref[1])
    h = jnp.concatenate([h0, h1], axis=1) + b_ref[...]
    h = jnp.maximum(h, 0.0)
    mu = jnp.dot(h, wmu_ref[...], preferred_element_type=jnp.float32,
                 precision=lax.Precision.HIGHEST) + bmu_ref[...]
    logvar = jnp.dot(h, wlv_ref[...], preferred_element_type=jnp.float32,
                     precision=lax.Precision.HIGHEST) + blv_ref[...]
    mu_ref[...] = mu
    lv_ref[...] = logvar
    lvc = jnp.clip(logvar, -20.0, 20.0)
    z = mu + eps_ref[...] * jnp.exp(0.5 * lvc)
    zpad_ref[...] = jnp.where(i < N // NB, z, 0.0)
    gids = lax.broadcasted_iota(jnp.int32, (G, N), 0)
    b = batch_ref[...]
    cnts_ref[...] = jnp.sum((b == gids).astype(jnp.int32), axis=1).reshape(1, G)
    offs_ref[...] = jnp.sum((b < gids).astype(jnp.int32), axis=1).reshape(1, G)


def _tc3(agg, hs, dinv, b2, wmu, bmu, wlv, blv, eps, batch2):
    clamp = lambda i: jnp.minimum(i, N // NB - 1)
    return pl.pallas_call(
        _tc3_body,
        grid=(ZP // NB,),
        in_specs=[
            pl.BlockSpec((NC, NB, HW), lambda i: (0, clamp(i), 0)),
            pl.BlockSpec((NC, NB, HW), lambda i: (0, clamp(i), 0)),
            pl.BlockSpec((NB, 1), lambda i: (clamp(i), 0)),
            pl.BlockSpec((1, D), lambda i: (0, 0)),
            pl.BlockSpec((D, LAT), lambda i: (0, 0)),
            pl.BlockSpec((1, LAT), lambda i: (0, 0)),
            pl.BlockSpec((D, LAT), lambda i: (0, 0)),
            pl.BlockSpec((1, LAT), lambda i: (0, 0)),
            pl.BlockSpec((NB, LAT), lambda i: (clamp(i), 0)),
            pl.BlockSpec((1, N), lambda i: (0, 0)),
        ],
        out_specs=[
            pl.BlockSpec((NB, LAT), lambda i: (clamp(i), 0)),
            pl.BlockSpec((NB, LAT), lambda i: (clamp(i), 0)),
            pl.BlockSpec((NB, LAT), lambda i: (i, 0)),
            pl.BlockSpec((1, G), lambda i: (0, 0)),
            pl.BlockSpec((1, G), lambda i: (0, 0)),
        ],
        out_shape=[
            jax.ShapeDtypeStruct((N, LAT), jnp.float32),
            jax.ShapeDtypeStruct((N, LAT), jnp.float32),
            jax.ShapeDtypeStruct((ZP, LAT), jnp.float32),
            jax.ShapeDtypeStruct((1, G), jnp.int32),
            jax.ShapeDtypeStruct((1, G), jnp.int32),
        ],
    )(agg, hs, dinv, b2, wmu, bmu, wlv, blv, eps, batch2)


def _dec_body(offs_ref, cnts_ref, z_ref, bias_ref, adj_ref, mask_ref):
    g = pl.program_id(0)
    off = offs_ref[g]
    cnt = cnts_ref[g]
    zb = z_ref[pl.ds(off, MN), :]
    colmask = lax.broadcasted_iota(jnp.int32, (MN, 1), 0) < cnt
    zm = jnp.where(colmask, zb, 0.0)
    logits = lax.dot_general(zm, zm, (((1,), (1,)), ((), ())),
                             preferred_element_type=jnp.float32,
                             precision=lax.Precision.HIGHEST)
    logits = logits * (LAT ** -0.5) + bias_ref[0]
    adj_ref[0] = jax.nn.sigmoid(logits)
    rowmask = lax.broadcasted_iota(jnp.int32, (1, MN), 1) < cnt
    mask_ref[0] = rowmask.astype(jnp.int32)


def _tc_decoder(offs, cnts, zpad, bias1):
    grid_spec = pltpu.PrefetchScalarGridSpec(
        num_scalar_prefetch=2,
        grid=(G,),
        in_specs=[
            pl.BlockSpec((ZP, LAT), lambda g, o, c: (0, 0)),
            pl.BlockSpec((1,), lambda g, o, c: (0,)),
        ],
        out_specs=[
            pl.BlockSpec((1, MN, MN), lambda g, o, c: (g, 0, 0)),
            pl.BlockSpec((1, 1, MN), lambda g, o, c: (g, 0, 0)),
        ],
    )
    return pl.pallas_call(
        _dec_body,
        grid_spec=grid_spec,
        out_shape=[
            jax.ShapeDtypeStruct((G, MN, MN), jnp.float32),
            jax.ShapeDtypeStruct((G, 1, MN), jnp.int32),
        ],
    )(offs, cnts, zpad, bias1)


def kernel(x, edge_index, batch, eps, W1, b1, W2, b2, Wmu, bmu, Wlv, blv,
           dec_bias):
    src = edge_index[0]
    dst = edge_index[1]
    npad = IRX * 128 - E
    srcp = jnp.concatenate([src, jnp.zeros((npad,), jnp.int32)])
    trash = N + jnp.arange(npad, dtype=jnp.int32) % NTRASH
    dstp = jnp.concatenate([dst, trash])
    dst3 = dstp.reshape(IRX, 128)
    srcs3 = jnp.stack([srcp, srcp + N], axis=0).reshape(NC, IRX, 128)
    ones128 = jnp.ones((128,), jnp.float32)
    zeros1 = jnp.zeros((CPT + TAIL,), jnp.float32)
    zerosC = jnp.zeros((CC, HW), jnp.float32)

    deg2 = _sc_degree(dst3, ones128, zeros1)

    w1s = jnp.stack([W1[:, :HW], W1[:, HW:]], axis=0)
    h1 = _tc_mm1(x + 0.0 * deg2[:N].reshape(N, 1), w1s)
    hs1, dinv = _tc_scale1(deg2.reshape(NC, N, 1), h1)
    agg1 = _sc_aggregate(hs1.reshape(NC * N, HW), srcs3, dst3, zerosC)

    hs2 = _tc2(agg1, hs1, dinv, b1.reshape(1, D), W2)
    agg2 = _sc_aggregate(hs2.reshape(NC * N, HW), srcs3, dst3, zerosC)

    mu, logvar, zpad, offs, cnts = _tc3(
        agg2, hs2, dinv, b2.reshape(1, D), Wmu, bmu.reshape(1, LAT),
        Wlv, blv.reshape(1, LAT), eps, batch.reshape(1, N))

    adj, maski = _tc_decoder(offs.reshape(G), cnts.reshape(G), zpad,
                             dec_bias.reshape(1))
    return adj, mu, logvar, maski.reshape(G, MN).astype(bool)


# speedup vs baseline: 1.6400x; 1.6388x over previous
import functools

import jax
import jax.numpy as jnp
from jax import lax
from jax.experimental import pallas as pl
from jax.experimental.pallas import tpu as pltpu
from jax.experimental.pallas import tpu_sc as plsc

N = 10000
E = 320000
D = 128
HW = 64
LAT = 64
G = 16
MN = 1024
NC, NS = 2, 16

IROWS = E // 128
IPAD = 2560
IRX = IPAD + 8
NTRASH = 512
DRPW = IPAD // (NC * NS)
DCH = 8
ARPT = IPAD // NS
ACH = 4
PCH = ARPT // ACH
CPT = 624
TAIL = N - CPT * NS
CC = 104

_SC_PARAMS = pltpu.CompilerParams(use_tc_tiling_on_sc=False)


def _sc_mesh():
    return plsc.VectorSubcoreMesh(core_axis_name="c", subcore_axis_name="s")


def _deg_body(dst_hbm, ones_hbm, zeros_hbm, out_hbm, dbuf, ones_v, stage, acc,
              sem):
    c = lax.axis_index("c")
    s = lax.axis_index("s")
    pltpu.sync_copy(zeros_hbm, stage)
    pltpu.sync_copy(stage.at[pl.ds(0, CPT)], acc.at[pl.ds(s * CPT, CPT)])

    @pl.when(s == NS - 1)
    def _():
        pltpu.sync_copy(stage.at[pl.ds(0, TAIL)],
                        acc.at[pl.ds(CPT * NS, TAIL)])

    pltpu.sync_copy(ones_hbm, ones_v)
    base = (c * NS + s) * DRPW
    pltpu.sync_copy(dst_hbm.at[pl.ds(base, DRPW), :], dbuf)
    plsc.subcore_barrier()

    ndch = jnp.where(s >= 0, DRPW // DCH, 0)

    def chunk(i, carry):
        descs = [
            pltpu.async_copy(ones_v, acc.at[dbuf.at[i * DCH + j]], sem,
                             add=True)
            for j in range(DCH)
        ]
        for d in descs:
            d.wait()
        return carry

    lax.fori_loop(0, ndch, chunk, 0)
    plsc.subcore_barrier()
    pltpu.sync_copy(acc.at[pl.ds(s * CPT, CPT)], stage.at[pl.ds(0, CPT)])
    pltpu.sync_copy(stage.at[pl.ds(0, CPT)],
                    out_hbm.at[pl.ds(c * N + s * CPT, CPT)])

    @pl.when(s == NS - 1)
    def _():
        pltpu.sync_copy(acc.at[pl.ds(CPT * NS, TAIL)],
                        stage.at[pl.ds(CPT, TAIL)])
        pltpu.sync_copy(stage.at[pl.ds(CPT, TAIL)],
                        out_hbm.at[pl.ds(c * N + CPT * NS, TAIL)])


def _sc_degree(dst3, ones128, zeros1):
    return pl.kernel(
        _deg_body,
        out_type=jax.ShapeDtypeStruct((NC * N,), jnp.float32),
        mesh=_sc_mesh(),
        scratch_types=[
            pltpu.VMEM((DRPW, 128), jnp.int32),
            pltpu.VMEM((128,), jnp.float32),
            pltpu.VMEM((CPT + TAIL,), jnp.float32),
            pltpu.VMEM_SHARED((N + NTRASH,), jnp.float32),
            pltpu.SemaphoreType.DMA,
        ],
        compiler_params=_SC_PARAMS,
    )(dst3, ones128, zeros1)


def _agg_body(hs_hbm, srcs_hbm, dst_hbm, zeros_hbm, out_hbm,
              sbufA, dbufA, rows, stage, acc, semA):
    c = lax.axis_index("c")
    s = lax.axis_index("s")
    pltpu.sync_copy(zeros_hbm, stage)
    for k in range(CPT // CC):
        pltpu.sync_copy(stage, acc.at[pl.ds(s * CPT + k * CC, CC)])

    @pl.when(s == NS - 1)
    def _():
        pltpu.sync_copy(stage.at[pl.ds(0, TAIL), :],
                        acc.at[pl.ds(CPT * NS, TAIL)])

    base = s * (IROWS // NS)
    plsc.subcore_barrier()
    nch = jnp.where(s == NS - 1, PCH, PCH - 1)

    def chunk(i, carry):
        rb = base + i * ACH
        pltpu.sync_copy(srcs_hbm.at[c, pl.ds(rb, ACH), :], sbufA)
        pltpu.sync_copy(dst_hbm.at[pl.ds(rb, ACH), :], dbufA)
        descs = [
            pltpu.async_copy(hs_hbm.at[sbufA.at[j]],
                             rows.at[pl.ds(j * 128, 128)], semA)
            for j in range(ACH)
        ]
        for d in descs:
            d.wait()
        for j in range(ACH):
            pltpu.sync_copy(rows.at[pl.ds(j * 128, 128)],
                            acc.at[dbufA.at[j]], add=True)
        return carry

    lax.fori_loop(0, nch, chunk, 0)
    plsc.subcore_barrier()
    for k in range(CPT // CC):
        pltpu.sync_copy(acc.at[pl.ds(s * CPT + k * CC, CC)], stage)
        pltpu.sync_copy(stage, out_hbm.at[c, pl.ds(s * CPT + k * CC, CC), :])

    @pl.when(s == NS - 1)
    def _():
        pltpu.sync_copy(acc.at[pl.ds(CPT * NS, TAIL)],
                        stage.at[pl.ds(0, TAIL), :])
        pltpu.sync_copy(stage.at[pl.ds(0, TAIL), :],
                        out_hbm.at[c, pl.ds(CPT * NS, TAIL), :])


def _sc_aggregate(hs_flat, srcs3, dst3, zerosC):
    return pl.kernel(
        _agg_body,
        out_type=jax.ShapeDtypeStruct((NC, N, HW), jnp.float32),
        mesh=_sc_mesh(),
        scratch_types=[
            pltpu.VMEM((ACH, 128), jnp.int32),
            pltpu.VMEM((ACH, 128), jnp.int32),
            pltpu.VMEM((ACH * 128, HW), jnp.float32),
            pltpu.VMEM((CC, HW), jnp.float32),
            pltpu.VMEM_SHARED((N + NTRASH, HW), jnp.float32),
            pltpu.SemaphoreType.DMA,
        ],
        compiler_params=_SC_PARAMS,
    )(hs_flat, srcs3, dst3, zerosC)


def _mm1_body(x_ref, w_ref, h_ref):
    h_ref[0] = jnp.dot(x_ref[...], w_ref[0], preferred_element_type=jnp.float32,
                       precision=lax.Precision.HIGHEST)


def _tc_mm1(x, w1s):
    return pl.pallas_call(
        _mm1_body,
        grid=(2,),
        in_specs=[
            pl.BlockSpec((N, D), lambda f: (0, 0)),
            pl.BlockSpec((1, D, HW), lambda f: (f, 0, 0)),
        ],
        out_specs=pl.BlockSpec((1, N, HW), lambda f: (f, 0, 0)),
        out_shape=jax.ShapeDtypeStruct((NC, N, HW), jnp.float32),
    )(x, w1s)


def _scale1_body(deg_ref, h_ref, hs_ref, dinv_ref):
    degsum = deg_ref[0] + deg_ref[1] + 1.0
    dinv = lax.rsqrt(degsum)
    dinv_ref[...] = dinv
    hs_ref[0] = h_ref[0] * dinv
    hs_ref[1] = h_ref[1] * dinv


def _tc_scale1(deg2, h1):
    return pl.pallas_call(
        _scale1_body,
        out_shape=[
            jax.ShapeDtypeStruct((NC, N, HW), jnp.float32),
            jax.ShapeDtypeStruct((N, 1), jnp.float32),
        ],
    )(deg2, h1)


def _tc2_body(agg_ref, hs_ref, dinv_ref, b_ref, w_ref, out_ref):
    dinv = dinv_ref[...]
    h0 = dinv * (agg_ref[0] + hs_ref[0])
    h1 = dinv * (agg_ref[1] + hs_ref[1])
    h = jnp.concatenate([h0, h1], axis=1) + b_ref[...]
    h = jnp.maximum(h, 0.0)
    h2 = jnp.dot(h, w_ref[...], preferred_element_type=jnp.float32,
                 precision=lax.Precision.HIGHEST)
    out_ref[0] = dinv * h2[:, :HW]
    out_ref[1] = dinv * h2[:, HW:]


NB = 2000


def _tc2(agg, hs, dinv, b1, w2):
    return pl.pallas_call(
        _tc2_body,
        grid=(N // NB,),
        in_specs=[
            pl.BlockSpec((NC, NB, HW), lambda i: (0, i, 0)),
            pl.BlockSpec((NC, NB, HW), lambda i: (0, i, 0)),
            pl.BlockSpec((NB, 1), lambda i: (i, 0)),
            pl.BlockSpec((1, D), lambda i: (0, 0)),
            pl.BlockSpec((D, D), lambda i: (0, 0)),
        ],
        out_specs=pl.BlockSpec((NC, NB, HW), lambda i: (0, i, 0)),
        out_shape=jax.ShapeDtypeStruct((NC, N, HW), jnp.float32),
    )(agg, hs, dinv, b1, w2)


ZP = N + NB


def _tc3_body(agg_ref, hs_ref, dinv_ref, b_ref, wmu_ref, bmu_ref, wlv_ref,
              blv_ref, eps_ref, batch_ref,
              mu_ref, lv_ref, zpad_ref, offs_ref, cnts_ref):
    i = pl.program_id(0)
    dinv = dinv_ref[...]
    h0 = dinv * (agg_ref[0] + hs_ref[0])
    h1 = dinv * (agg_ref[1] + hs_ref[1])
    h = jnp.concatenate([h0, h1], axis=1) + b_ref[...]
    h = jnp.maximum(h, 0.0)
    mu = jnp.dot(h, wmu_ref[...], preferred_element_type=jnp.float32,
                 precision=lax.Precision.HIGHEST) + bmu_ref[...]
    logvar = jnp.dot(h, wlv_ref[...], preferred_element_type=jnp.float32,
                     precision=lax.Precision.HIGHEST) + blv_ref[...]
    mu_ref[...] = mu
    lv_ref[...] = logvar
    lvc = jnp.clip(logvar, -20.0, 20.0)
    z = mu + eps_ref[...] * jnp.exp(0.5 * lvc)
    zpad_ref[...] = jnp.where(i < N // NB, z, 0.0)
    gids = lax.broadcasted_iota(jnp.int32, (G, N), 0)
    b = batch_ref[...]
    cnts_ref[...] = jnp.sum((b == gids).astype(jnp.int32), axis=1).reshape(1, G)
    offs_ref[...] = jnp.sum((b < gids).astype(jnp.int32), axis=1).reshape(1, G)


def _tc3(agg, hs, dinv, b2, wmu, bmu, wlv, blv, eps, batch2):
    clamp = lambda i: jnp.minimum(i, N // NB - 1)
    return pl.pallas_call(
        _tc3_body,
        grid=(ZP // NB,),
        in_specs=[
            pl.BlockSpec((NC, NB, HW), lambda i: (0, clamp(i), 0)),
            pl.BlockSpec((NC, NB, HW), lambda i: (0, clamp(i), 0)),
            pl.BlockSpec((NB, 1), lambda i: (clamp(i), 0)),
            pl.BlockSpec((1, D), lambda i: (0, 0)),
            pl.BlockSpec((D, LAT), lambda i: (0, 0)),
            pl.BlockSpec((1, LAT), lambda i: (0, 0)),
            pl.BlockSpec((D, LAT), lambda i: (0, 0)),
            pl.BlockSpec((1, LAT), lambda i: (0, 0)),
            pl.BlockSpec((NB, LAT), lambda i: (clamp(i), 0)),
            pl.BlockSpec((1, N), lambda i: (0, 0)),
        ],
        out_specs=[
            pl.BlockSpec((NB, LAT), lambda i: (clamp(i), 0)),
            pl.BlockSpec((NB, LAT), lambda i: (clamp(i), 0)),
            pl.BlockSpec((NB, LAT), lambda i: (i, 0)),
            pl.BlockSpec((1, G), lambda i: (0, 0)),
            pl.BlockSpec((1, G), lambda i: (0, 0)),
        ],
        out_shape=[
            jax.ShapeDtypeStruct((N, LAT), jnp.float32),
            jax.ShapeDtypeStruct((N, LAT), jnp.float32),
            jax.ShapeDtypeStruct((ZP, LAT), jnp.float32),
            jax.ShapeDtypeStruct((1, G), jnp.int32),
            jax.ShapeDtypeStruct((1, G), jnp.int32),
        ],
    )(agg, hs, dinv, b2, wmu, bmu, wlv, blv, eps, batch2)


def _dec_body(offs_ref, cnts_ref, z_ref, bias_ref, adj_ref, mask_ref):
    g = pl.program_id(0)
    off = offs_ref[g]
    cnt = cnts_ref[g]
    zb = z_ref[pl.ds(off, MN), :]
    colmask = lax.broadcasted_iota(jnp.int32, (MN, 1), 0) < cnt
    zm = jnp.where(colmask, zb, 0.0)
    logits = lax.dot_general(zm, zm, (((1,), (1,)), ((), ())),
                             preferred_element_type=jnp.float32,
                             precision=lax.Precision.HIGHEST)
    logits = logits * (LAT ** -0.5) + bias_ref[0]
    adj_ref[0] = jax.nn.sigmoid(logits)
    rowmask = lax.broadcasted_iota(jnp.int32, (1, MN), 1) < cnt
    mask_ref[0] = rowmask.astype(jnp.int32)


def _tc_decoder(offs, cnts, zpad, bias1):
    grid_spec = pltpu.PrefetchScalarGridSpec(
        num_scalar_prefetch=2,
        grid=(G,),
        in_specs=[
            pl.BlockSpec((ZP, LAT), lambda g, o, c: (0, 0)),
            pl.BlockSpec((1,), lambda g, o, c: (0,)),
        ],
        out_specs=[
            pl.BlockSpec((1, MN, MN), lambda g, o, c: (g, 0, 0)),
            pl.BlockSpec((1, 1, MN), lambda g, o, c: (g, 0, 0)),
        ],
    )
    return pl.pallas_call(
        _dec_body,
        grid_spec=grid_spec,
        out_shape=[
            jax.ShapeDtypeStruct((G, MN, MN), jnp.float32),
            jax.ShapeDtypeStruct((G, 1, MN), jnp.int32),
        ],
    )(offs, cnts, zpad, bias1)


def kernel(x, edge_index, batch, eps, W1, b1, W2, b2, Wmu, bmu, Wlv, blv,
           dec_bias):
    src = edge_index[0]
    dst = edge_index[1]
    npad = IRX * 128 - E
    srcp = jnp.concatenate([src, jnp.zeros((npad,), jnp.int32)])
    trash = N + jnp.arange(npad, dtype=jnp.int32) % NTRASH
    dstp = jnp.concatenate([dst, trash])
    dst3 = dstp.reshape(IRX, 128)
    srcs3 = jnp.stack([srcp, srcp + N], axis=0).reshape(NC, IRX, 128)
    ones128 = jnp.ones((128,), jnp.float32)
    zeros1 = jnp.zeros((CPT + TAIL,), jnp.float32)
    zerosC = jnp.zeros((CC, HW), jnp.float32)

    deg2 = _sc_degree(dst3, ones128, zeros1)

    w1s = jnp.stack([W1[:, :HW], W1[:, HW:]], axis=0)
    h1 = _tc_mm1(x + 0.0 * deg2[:N].reshape(N, 1), w1s)
    hs1, dinv = _tc_scale1(deg2.reshape(NC, N, 1), h1)
    agg1 = _sc_aggregate(hs1.reshape(NC * N, HW), srcs3, dst3, zerosC)

    hs2 = _tc2(agg1, hs1, dinv, b1.reshape(1, D), W2)
    agg2 = _sc_aggregate(hs2.reshape(NC * N, HW), srcs3, dst3, zerosC)

    mu, logvar, zpad, offs, cnts = _tc3(
        agg2, hs2, dinv, b2.reshape(1, D), Wmu, bmu.reshape(1, LAT),
        Wlv, blv.reshape(1, LAT), eps, batch.reshape(1, N))

    adj, maski = _tc_decoder(offs.reshape(G), cnts.reshape(G), zpad,
                             dec_bias.reshape(1))
    return adj, mu, logvar, maski.reshape(G, MN).astype(bool)


# speedup vs baseline: 1.6697x; 1.0181x over previous
import functools

import jax
import jax.numpy as jnp
from jax import lax
from jax.experimental import pallas as pl
from jax.experimental.pallas import tpu as pltpu
from jax.experimental.pallas import tpu_sc as plsc

N = 10000
E = 320000
D = 128
HW = 64
LAT = 64
G = 16
MN = 1024
NC, NS = 2, 16

IROWS = E // 128
IPAD = 2560
IRX = IPAD + 8
NTRASH = 512
DRPW = IPAD // (NC * NS)
DCH = 8
ARPT = IPAD // NS
ACH = 4
PCH = ARPT // ACH
CPT = 624
TAIL = N - CPT * NS
CC = 104

_SC_PARAMS = pltpu.CompilerParams(use_tc_tiling_on_sc=False)


def _sc_mesh():
    return plsc.VectorSubcoreMesh(core_axis_name="c", subcore_axis_name="s")


def _deg_body(dst_hbm, ones_hbm, zeros_hbm, out_hbm, dbuf, ones_v, stage, acc,
              sem):
    c = lax.axis_index("c")
    s = lax.axis_index("s")
    pltpu.sync_copy(zeros_hbm, stage)
    pltpu.sync_copy(stage.at[pl.ds(0, CPT)], acc.at[pl.ds(s * CPT, CPT)])

    @pl.when(s == NS - 1)
    def _():
        pltpu.sync_copy(stage.at[pl.ds(0, TAIL)],
                        acc.at[pl.ds(CPT * NS, TAIL)])

    pltpu.sync_copy(ones_hbm, ones_v)
    base = (c * NS + s) * DRPW
    pltpu.sync_copy(dst_hbm.at[pl.ds(base, DRPW), :], dbuf)
    plsc.subcore_barrier()

    ndch = jnp.where(s >= 0, DRPW // DCH, 0)

    def chunk(i, carry):
        descs = [
            pltpu.async_copy(ones_v, acc.at[dbuf.at[i * DCH + j]], sem,
                             add=True)
            for j in range(DCH)
        ]
        for d in descs:
            d.wait()
        return carry

    lax.fori_loop(0, ndch, chunk, 0)
    plsc.subcore_barrier()
    pltpu.sync_copy(acc.at[pl.ds(s * CPT, CPT)], stage.at[pl.ds(0, CPT)])
    pltpu.sync_copy(stage.at[pl.ds(0, CPT)],
                    out_hbm.at[pl.ds(c * N + s * CPT, CPT)])

    @pl.when(s == NS - 1)
    def _():
        pltpu.sync_copy(acc.at[pl.ds(CPT * NS, TAIL)],
                        stage.at[pl.ds(CPT, TAIL)])
        pltpu.sync_copy(stage.at[pl.ds(CPT, TAIL)],
                        out_hbm.at[pl.ds(c * N + CPT * NS, TAIL)])


def _sc_degree(dst3, ones128, zeros1):
    return pl.kernel(
        _deg_body,
        out_type=jax.ShapeDtypeStruct((NC * N,), jnp.float32),
        mesh=_sc_mesh(),
        scratch_types=[
            pltpu.VMEM((DRPW, 128), jnp.int32),
            pltpu.VMEM((128,), jnp.float32),
            pltpu.VMEM((CPT + TAIL,), jnp.float32),
            pltpu.VMEM_SHARED((N + NTRASH,), jnp.float32),
            pltpu.SemaphoreType.DMA,
        ],
        compiler_params=_SC_PARAMS,
    )(dst3, ones128, zeros1)


def _agg_body(hs_hbm, srcs_hbm, dst_hbm, zeros_hbm, out_hbm,
              sbufA, dbufA, rows, stage, acc, semA):
    c = lax.axis_index("c")
    s = lax.axis_index("s")
    pltpu.sync_copy(zeros_hbm, stage)
    for k in range(CPT // CC):
        pltpu.sync_copy(stage, acc.at[pl.ds(s * CPT + k * CC, CC)])

    @pl.when(s == NS - 1)
    def _():
        pltpu.sync_copy(stage.at[pl.ds(0, TAIL), :],
                        acc.at[pl.ds(CPT * NS, TAIL)])

    base = s * (IROWS // NS)
    plsc.subcore_barrier()
    nch = jnp.where(s == NS - 1, PCH, PCH - 1)

    def chunk(i, carry):
        rb = base + i * ACH
        pltpu.sync_copy(srcs_hbm.at[c, pl.ds(rb, ACH), :], sbufA)
        pltpu.sync_copy(dst_hbm.at[pl.ds(rb, ACH), :], dbufA)
        descs = [
            pltpu.async_copy(hs_hbm.at[sbufA.at[j]],
                             rows.at[pl.ds(j * 128, 128)], semA)
            for j in range(ACH)
        ]
        for d in descs:
            d.wait()
        for j in range(ACH):
            pltpu.sync_copy(rows.at[pl.ds(j * 128, 128)],
                            acc.at[dbufA.at[j]], add=True)
        return carry

    lax.fori_loop(0, nch, chunk, 0)
    plsc.subcore_barrier()
    for k in range(CPT // CC):
        pltpu.sync_copy(acc.at[pl.ds(s * CPT + k * CC, CC)], stage)
        pltpu.sync_copy(stage, out_hbm.at[c, pl.ds(s * CPT + k * CC, CC), :])

    @pl.when(s == NS - 1)
    def _():
        pltpu.sync_copy(acc.at[pl.ds(CPT * NS, TAIL)],
                        stage.at[pl.ds(0, TAIL), :])
        pltpu.sync_copy(stage.at[pl.ds(0, TAIL), :],
                        out_hbm.at[c, pl.ds(CPT * NS, TAIL), :])


def _sc_aggregate(hs_flat, srcs3, dst3, zerosC):
    return pl.kernel(
        _agg_body,
        out_type=jax.ShapeDtypeStruct((NC, N, HW), jnp.float32),
        mesh=_sc_mesh(),
        scratch_types=[
            pltpu.VMEM((ACH, 128), jnp.int32),
            pltpu.VMEM((ACH, 128), jnp.int32),
            pltpu.VMEM((ACH * 128, HW), jnp.float32),
            pltpu.VMEM((CC, HW), jnp.float32),
            pltpu.VMEM_SHARED((N + NTRASH, HW), jnp.float32),
            pltpu.SemaphoreType.DMA,
        ],
        compiler_params=_SC_PARAMS,
    )(hs_flat, srcs3, dst3, zerosC)


def _mm1_body(x_ref, w_ref, h_ref):
    h_ref[0] = jnp.dot(x_ref[...], w_ref[0], preferred_element_type=jnp.float32,
                       precision=lax.Precision.HIGHEST)


def _tc_mm1(x, w1s):
    return pl.pallas_call(
        _mm1_body,
        grid=(2,),
        in_specs=[
            pl.BlockSpec((N, D), lambda f: (0, 0)),
            pl.BlockSpec((1, D, HW), lambda f: (f, 0, 0)),
        ],
        out_specs=pl.BlockSpec((1, N, HW), lambda f: (f, 0, 0)),
        out_shape=jax.ShapeDtypeStruct((NC, N, HW), jnp.float32),
    )(x, w1s)


def _scale1_body(deg_ref, h_ref, hs_ref, dinv_ref):
    degsum = deg_ref[0] + deg_ref[1] + 1.0
    dinv = lax.rsqrt(degsum)
    dinv_ref[...] = dinv
    hs_ref[0] = h_ref[0] * dinv
    hs_ref[1] = h_ref[1] * dinv


def _tc_scale1(deg2, h1):
    return pl.pallas_call(
        _scale1_body,
        out_shape=[
            jax.ShapeDtypeStruct((NC, N, HW), jnp.float32),
            jax.ShapeDtypeStruct((N, 1), jnp.float32),
        ],
    )(deg2, h1)


def _tc2_body(agg_ref, hs_ref, dinv_ref, b_ref, w_ref, out_ref):
    dinv = dinv_ref[...]
    h0 = dinv * (agg_ref[0] + hs_ref[0])
    h1 = dinv * (agg_ref[1] + hs_ref[1])
    h = jnp.concatenate([h0, h1], axis=1) + b_ref[...]
    h = jnp.maximum(h, 0.0)
    h2 = jnp.dot(h, w_ref[...], preferred_element_type=jnp.float32,
                 precision=lax.Precision.HIGHEST)
    out_ref[0] = dinv * h2[:, :HW]
    out_ref[1] = dinv * h2[:, HW:]


NB = 2000


def _tc2(agg, hs, dinv, b1, w2):
    return pl.pallas_call(
        _tc2_body,
        grid=(N // NB,),
        in_specs=[
            pl.BlockSpec((NC, NB, HW), lambda i: (0, i, 0)),
            pl.BlockSpec((NC, NB, HW), lambda i: (0, i, 0)),
            pl.BlockSpec((NB, 1), lambda i: (i, 0)),
            pl.BlockSpec((1, D), lambda i: (0, 0)),
            pl.BlockSpec((D, D), lambda i: (0, 0)),
        ],
        out_specs=pl.BlockSpec((NC, NB, HW), lambda i: (0, i, 0)),
        out_shape=jax.ShapeDtypeStruct((NC, N, HW), jnp.float32),
    )(agg, hs, dinv, b1, w2)


ZP = N + NB


def _tc3_body(agg_ref, hs_ref, dinv_ref, b_ref, wmu_ref, bmu_ref, wlv_ref,
              blv_ref, eps_ref, batch_ref,
              mu_ref, lv_ref, zpad_ref, offs_ref, cnts_ref):
    i = pl.program_id(0)
    dinv = dinv_ref[...]
    h0 = dinv * (agg_ref[0] + hs_ref[0])
    h1 = dinv * (agg_ref[1] + hs_ref[1])
    h = jnp.concatenate([h0, h1], axis=1) + b_ref[...]
    h = jnp.maximum(h, 0.0)
    mu = jnp.dot(h, wmu_ref[...], preferred_element_type=jnp.float32,
                 precision=lax.Precision.HIGHEST) + bmu_ref[...]
    logvar = jnp.dot(h, wlv_ref[...], preferred_element_type=jnp.float32,
                     precision=lax.Precision.HIGHEST) + blv_ref[...]
    mu_ref[...] = mu
    lv_ref[...] = logvar
    lvc = jnp.clip(logvar, -20.0, 20.0)
    z = mu + eps_ref[...] * jnp.exp(0.5 * lvc)
    zpad_ref[...] = jnp.where(i < N // NB, z, 0.0)
    gids = lax.broadcasted_iota(jnp.int32, (G, N), 0)
    b = batch_ref[...]
    cnts_ref[...] = jnp.sum((b == gids).astype(jnp.int32), axis=1).reshape(1, G)
    offs_ref[...] = jnp.sum((b < gids).astype(jnp.int32), axis=1).reshape(1, G)


def _tc3(agg, hs, dinv, b2, wmu, bmu, wlv, blv, eps, batch2):
    clamp = lambda i: jnp.minimum(i, N // NB - 1)
    return pl.pallas_call(
        _tc3_body,
        grid=(ZP // NB,),
        in_specs=[
            pl.BlockSpec((NC, NB, HW), lambda i: (0, clamp(i), 0)),
            pl.BlockSpec((NC, NB, HW), lambda i: (0, clamp(i), 0)),
            pl.BlockSpec((NB, 1), lambda i: (clamp(i), 0)),
            pl.BlockSpec((1, D), lambda i: (0, 0)),
            pl.BlockSpec((D, LAT), lambda i: (0, 0)),
            pl.BlockSpec((1, LAT), lambda i: (0, 0)),
            pl.BlockSpec((D, LAT), lambda i: (0, 0)),
            pl.BlockSpec((1, LAT), lambda i: (0, 0)),
            pl.BlockSpec((NB, LAT), lambda i: (clamp(i), 0)),
            pl.BlockSpec((1, N), lambda i: (0, 0)),
        ],
        out_specs=[
            pl.BlockSpec((NB, LAT), lambda i: (clamp(i), 0)),
            pl.BlockSpec((NB, LAT), lambda i: (clamp(i), 0)),
            pl.BlockSpec((NB, LAT), lambda i: (i, 0)),
            pl.BlockSpec((1, G), lambda i: (0, 0)),
            pl.BlockSpec((1, G), lambda i: (0, 0)),
        ],
        out_shape=[
            jax.ShapeDtypeStruct((N, LAT), jnp.float32),
            jax.ShapeDtypeStruct((N, LAT), jnp.float32),
            jax.ShapeDtypeStruct((ZP, LAT), jnp.float32),
            jax.ShapeDtypeStruct((1, G), jnp.int32),
            jax.ShapeDtypeStruct((1, G), jnp.int32),
        ],
    )(agg, hs, dinv, b2, wmu, bmu, wlv, blv, eps, batch2)


def _dec_body(offs_ref, cnts_ref, z_ref, bias_ref, adj_ref, mask_ref):
    g = pl.program_id(0)
    off = offs_ref[g]
    cnt = cnts_ref[g]
    zb = z_ref[pl.ds(off, MN), :]
    colmask = lax.broadcasted_iota(jnp.int32, (MN, 1), 0) < cnt
    zm = jnp.where(colmask, zb, 0.0)
    logits = lax.dot_general(zm, zm, (((1,), (1,)), ((), ())),
                             preferred_element_type=jnp.float32,
                             precision=lax.Precision.HIGHEST)
    logits = logits * (LAT ** -0.5) + bias_ref[0]
    adj_ref[0] = jax.nn.sigmoid(logits)
    rowmask = lax.broadcasted_iota(jnp.int32, (1, MN), 1) < cnt
    mask_ref[0] = rowmask.astype(jnp.int32)


def _tc_decoder(offs, cnts, zpad, bias1):
    grid_spec = pltpu.PrefetchScalarGridSpec(
        num_scalar_prefetch=2,
        grid=(G,),
        in_specs=[
            pl.BlockSpec((ZP, LAT), lambda g, o, c: (0, 0)),
            pl.BlockSpec((1,), lambda g, o, c: (0,)),
        ],
        out_specs=[
            pl.BlockSpec((1, MN, MN), lambda g, o, c: (g, 0, 0)),
            pl.BlockSpec((1, 1, MN), lambda g, o, c: (g, 0, 0)),
        ],
    )
    return pl.pallas_call(
        _dec_body,
        grid_spec=grid_spec,
        out_shape=[
            jax.ShapeDtypeStruct((G, MN, MN), jnp.float32),
            jax.ShapeDtypeStruct((G, 1, MN), jnp.int32),
        ],
    )(offs, cnts, zpad, bias1)


def kernel(x, edge_index, batch, eps, W1, b1, W2, b2, Wmu, bmu, Wlv, blv,
           dec_bias):
    src = edge_index[0]
    dst = edge_index[1]
    npad = IRX * 128 - E
    srcp = jnp.concatenate([src, jnp.zeros((npad,), jnp.int32)])
    trash = N + jnp.arange(npad, dtype=jnp.int32) % NTRASH
    dstp = jnp.concatenate([dst, trash])
    dst3 = dstp.reshape(IRX, 128)
    srcs3 = jnp.stack([srcp, srcp + N], axis=0).reshape(NC, IRX, 128)
    ones128 = jnp.ones((128,), jnp.float32)
    zeros1 = jnp.zeros((CPT + TAIL,), jnp.float32)
    zerosC = jnp.zeros((CC, HW), jnp.float32)

    deg2 = _sc_degree(dst3, ones128, zeros1)

    w1s = jnp.stack([W1[:, :HW], W1[:, HW:]], axis=0)
    h1 = _tc_mm1(x, w1s)
    hs1, dinv = _tc_scale1(deg2.reshape(NC, N, 1), h1)
    agg1 = _sc_aggregate(hs1.reshape(NC * N, HW), srcs3, dst3, zerosC)

    hs2 = _tc2(agg1, hs1, dinv, b1.reshape(1, D), W2)
    agg2 = _sc_aggregate(hs2.reshape(NC * N, HW), srcs3, dst3, zerosC)

    mu, logvar, zpad, offs, cnts = _tc3(
        agg2, hs2, dinv, b2.reshape(1, D), Wmu, bmu.reshape(1, LAT),
        Wlv, blv.reshape(1, LAT), eps, batch.reshape(1, N))

    adj, maski = _tc_decoder(offs.reshape(G), cnts.reshape(G), zpad,
                             dec_bias.reshape(1))
    return adj, mu, logvar, maski.reshape(G, MN).astype(bool)


# speedup vs baseline: 2.0851x; 1.2488x over previous
import functools

import jax
import jax.numpy as jnp
from jax import lax
from jax.experimental import pallas as pl
from jax.experimental.pallas import tpu as pltpu
from jax.experimental.pallas import tpu_sc as plsc

N = 10000
E = 320000
D = 128
HW = 64
LAT = 64
G = 16
MN = 1024
NC, NS = 2, 16

IROWS = E // 128
IPAD = 2560
IRX = IPAD + 8
NTRASH = 512
DRPW = IPAD // (NC * NS)
DCH = 8
ARPT = IPAD // NS
ACH = 4
PCH = ARPT // ACH
CPT = 624
TAIL = N - CPT * NS
CC = 104

_SC_PARAMS = pltpu.CompilerParams(use_tc_tiling_on_sc=False)


def _sc_mesh():
    return plsc.VectorSubcoreMesh(core_axis_name="c", subcore_axis_name="s")


def _deg_body(dst_hbm, ones_hbm, zeros_hbm, out_hbm, dbuf, ones_v, stage, acc,
              sem):
    c = lax.axis_index("c")
    s = lax.axis_index("s")
    pltpu.sync_copy(zeros_hbm, stage)
    pltpu.sync_copy(stage.at[pl.ds(0, CPT)], acc.at[pl.ds(s * CPT, CPT)])

    @pl.when(s == NS - 1)
    def _():
        pltpu.sync_copy(stage.at[pl.ds(0, TAIL)],
                        acc.at[pl.ds(CPT * NS, TAIL)])

    pltpu.sync_copy(ones_hbm, ones_v)
    base = (c * NS + s) * DRPW
    pltpu.sync_copy(dst_hbm.at[pl.ds(base, DRPW), :], dbuf)
    plsc.subcore_barrier()

    ndch = jnp.where(s >= 0, DRPW // DCH, 0)

    def chunk(i, carry):
        descs = [
            pltpu.async_copy(ones_v, acc.at[dbuf.at[i * DCH + j]], sem,
                             add=True)
            for j in range(DCH)
        ]
        for d in descs:
            d.wait()
        return carry

    lax.fori_loop(0, ndch, chunk, 0)
    plsc.subcore_barrier()
    pltpu.sync_copy(acc.at[pl.ds(s * CPT, CPT)], stage.at[pl.ds(0, CPT)])
    pltpu.sync_copy(stage.at[pl.ds(0, CPT)],
                    out_hbm.at[pl.ds(c * N + s * CPT, CPT)])

    @pl.when(s == NS - 1)
    def _():
        pltpu.sync_copy(acc.at[pl.ds(CPT * NS, TAIL)],
                        stage.at[pl.ds(CPT, TAIL)])
        pltpu.sync_copy(stage.at[pl.ds(CPT, TAIL)],
                        out_hbm.at[pl.ds(c * N + CPT * NS, TAIL)])


def _sc_degree(dst3, ones128, zeros1):
    return pl.kernel(
        _deg_body,
        out_type=jax.ShapeDtypeStruct((NC * N,), jnp.float32),
        mesh=_sc_mesh(),
        scratch_types=[
            pltpu.VMEM((DRPW, 128), jnp.int32),
            pltpu.VMEM((128,), jnp.float32),
            pltpu.VMEM((CPT + TAIL,), jnp.float32),
            pltpu.VMEM_SHARED((N + NTRASH,), jnp.float32),
            pltpu.SemaphoreType.DMA,
        ],
        compiler_params=_SC_PARAMS,
    )(dst3, ones128, zeros1)


def _agg_body(hs_hbm, srcs_hbm, dst_hbm, zeros_hbm, out_hbm,
              sbufA, dbufA, sbufB, dbufB, rowsA, rowsB, stage, acc,
              semA, semB):
    c = lax.axis_index("c")
    s = lax.axis_index("s")
    pltpu.sync_copy(zeros_hbm, stage)
    for k in range(CPT // CC):
        pltpu.sync_copy(stage, acc.at[pl.ds(s * CPT + k * CC, CC)])

    @pl.when(s == NS - 1)
    def _():
        pltpu.sync_copy(stage.at[pl.ds(0, TAIL), :],
                        acc.at[pl.ds(CPT * NS, TAIL)])

    base = s * ARPT
    plsc.subcore_barrier()

    def idx_load(sbuf, dbuf, ch):
        rb = base + ch * ACH
        pltpu.sync_copy(srcs_hbm.at[c, pl.ds(rb, ACH), :], sbuf)
        pltpu.sync_copy(dst_hbm.at[pl.ds(rb, ACH), :], dbuf)

    def fire(sbuf, rows, sem):
        for j in range(ACH):
            pltpu.async_copy(hs_hbm.at[sbuf.at[j]],
                             rows.at[pl.ds(j * 128, 128)], sem)

    def gather_wait(sbuf, rows, sem):
        for j in range(ACH):
            pltpu.make_async_copy(hs_hbm.at[sbuf.at[j]],
                                  rows.at[pl.ds(j * 128, 128)], sem).wait()

    def scatter(dbuf, rows):
        for j in range(ACH):
            pltpu.sync_copy(rows.at[pl.ds(j * 128, 128)],
                            acc.at[dbuf.at[j]], add=True)

    idx_load(sbufA, dbufA, 0)
    fire(sbufA, rowsA, semA)
    npair = jnp.where(s >= 0, PCH // 2, 0)

    def body(p, carry):
        a = 2 * p
        idx_load(sbufB, dbufB, a + 1)
        gather_wait(sbufA, rowsA, semA)
        fire(sbufB, rowsB, semB)
        scatter(dbufA, rowsA)
        idx_load(sbufA, dbufA, a + 2)
        gather_wait(sbufB, rowsB, semB)
        fire(sbufA, rowsA, semA)
        scatter(dbufB, rowsB)
        return carry

    lax.fori_loop(0, npair, body, 0)
    gather_wait(sbufA, rowsA, semA)
    plsc.subcore_barrier()
    for k in range(CPT // CC):
        pltpu.sync_copy(acc.at[pl.ds(s * CPT + k * CC, CC)], stage)
        pltpu.sync_copy(stage, out_hbm.at[c, pl.ds(s * CPT + k * CC, CC), :])

    @pl.when(s == NS - 1)
    def _():
        pltpu.sync_copy(acc.at[pl.ds(CPT * NS, TAIL)],
                        stage.at[pl.ds(0, TAIL), :])
        pltpu.sync_copy(stage.at[pl.ds(0, TAIL), :],
                        out_hbm.at[c, pl.ds(CPT * NS, TAIL), :])


def _sc_aggregate(hs_flat, srcs3, dst3, zerosC):
    return pl.kernel(
        _agg_body,
        out_type=jax.ShapeDtypeStruct((NC, N, HW), jnp.float32),
        mesh=_sc_mesh(),
        scratch_types=[
            pltpu.VMEM((ACH, 128), jnp.int32),
            pltpu.VMEM((ACH, 128), jnp.int32),
            pltpu.VMEM((ACH, 128), jnp.int32),
            pltpu.VMEM((ACH, 128), jnp.int32),
            pltpu.VMEM((ACH * 128, HW), jnp.float32),
            pltpu.VMEM((ACH * 128, HW), jnp.float32),
            pltpu.VMEM((CC, HW), jnp.float32),
            pltpu.VMEM_SHARED((N + NTRASH, HW), jnp.float32),
            pltpu.SemaphoreType.DMA,
            pltpu.SemaphoreType.DMA,
        ],
        compiler_params=_SC_PARAMS,
    )(hs_flat, srcs3, dst3, zerosC)


def _mm1_body(x_ref, w_ref, h_ref):
    h_ref[0] = jnp.dot(x_ref[...], w_ref[0], preferred_element_type=jnp.float32,
                       precision=lax.Precision.HIGHEST)


def _tc_mm1(x, w1s):
    return pl.pallas_call(
        _mm1_body,
        grid=(2,),
        in_specs=[
            pl.BlockSpec((N, D), lambda f: (0, 0)),
            pl.BlockSpec((1, D, HW), lambda f: (f, 0, 0)),
        ],
        out_specs=pl.BlockSpec((1, N, HW), lambda f: (f, 0, 0)),
        out_shape=jax.ShapeDtypeStruct((NC, N, HW), jnp.float32),
    )(x, w1s)


def _scale1_body(deg_ref, h_ref, hs_ref, dinv_ref):
    degsum = deg_ref[0] + deg_ref[1] + 1.0
    dinv = lax.rsqrt(degsum)
    dinv_ref[...] = dinv
    hs_ref[0] = h_ref[0] * dinv
    hs_ref[1] = h_ref[1] * dinv


def _tc_scale1(deg2, h1):
    return pl.pallas_call(
        _scale1_body,
        out_shape=[
            jax.ShapeDtypeStruct((NC, N, HW), jnp.float32),
            jax.ShapeDtypeStruct((N, 1), jnp.float32),
        ],
    )(deg2, h1)


def _tc2_body(agg_ref, hs_ref, dinv_ref, b_ref, w_ref, out_ref):
    dinv = dinv_ref[...]
    h0 = dinv * (agg_ref[0] + hs_ref[0])
    h1 = dinv * (agg_ref[1] + hs_ref[1])
    h = jnp.concatenate([h0, h1], axis=1) + b_ref[...]
    h = jnp.maximum(h, 0.0)
    h2 = jnp.dot(h, w_ref[...], preferred_element_type=jnp.float32,
                 precision=lax.Precision.HIGHEST)
    out_ref[0] = dinv * h2[:, :HW]
    out_ref[1] = dinv * h2[:, HW:]


NB = 2000


def _tc2(agg, hs, dinv, b1, w2):
    return pl.pallas_call(
        _tc2_body,
        grid=(N // NB,),
        in_specs=[
            pl.BlockSpec((NC, NB, HW), lambda i: (0, i, 0)),
            pl.BlockSpec((NC, NB, HW), lambda i: (0, i, 0)),
            pl.BlockSpec((NB, 1), lambda i: (i, 0)),
            pl.BlockSpec((1, D), lambda i: (0, 0)),
            pl.BlockSpec((D, D), lambda i: (0, 0)),
        ],
        out_specs=pl.BlockSpec((NC, NB, HW), lambda i: (0, i, 0)),
        out_shape=jax.ShapeDtypeStruct((NC, N, HW), jnp.float32),
    )(agg, hs, dinv, b1, w2)


ZP = N + NB


def _tc3_body(agg_ref, hs_ref, dinv_ref, b_ref, wmu_ref, bmu_ref, wlv_ref,
              blv_ref, eps_ref, batch_ref,
              mu_ref, lv_ref, zpad_ref, offs_ref, cnts_ref):
    i = pl.program_id(0)
    dinv = dinv_ref[...]
    h0 = dinv * (agg_ref[0] + hs_ref[0])
    h1 = dinv * (agg_ref[1] + hs_ref[1])
    h = jnp.concatenate([h0, h1], axis=1) + b_ref[...]
    h = jnp.maximum(h, 0.0)
    mu = jnp.dot(h, wmu_ref[...], preferred_element_type=jnp.float32,
                 precision=lax.Precision.HIGHEST) + bmu_ref[...]
    logvar = jnp.dot(h, wlv_ref[...], preferred_element_type=jnp.float32,
                     precision=lax.Precision.HIGHEST) + blv_ref[...]
    mu_ref[...] = mu
    lv_ref[...] = logvar
    lvc = jnp.clip(logvar, -20.0, 20.0)
    z = mu + eps_ref[...] * jnp.exp(0.5 * lvc)
    zpad_ref[...] = jnp.where(i < N // NB, z, 0.0)
    gids = lax.broadcasted_iota(jnp.int32, (G, N), 0)
    b = batch_ref[...]
    cnts_ref[...] = jnp.sum((b == gids).astype(jnp.int32), axis=1).reshape(1, G)
    offs_ref[...] = jnp.sum((b < gids).astype(jnp.int32), axis=1).reshape(1, G)


def _tc3(agg, hs, dinv, b2, wmu, bmu, wlv, blv, eps, batch2):
    clamp = lambda i: jnp.minimum(i, N // NB - 1)
    return pl.pallas_call(
        _tc3_body,
        grid=(ZP // NB,),
        in_specs=[
            pl.BlockSpec((NC, NB, HW), lambda i: (0, clamp(i), 0)),
            pl.BlockSpec((NC, NB, HW), lambda i: (0, clamp(i), 0)),
            pl.BlockSpec((NB, 1), lambda i: (clamp(i), 0)),
            pl.BlockSpec((1, D), lambda i: (0, 0)),
            pl.BlockSpec((D, LAT), lambda i: (0, 0)),
            pl.BlockSpec((1, LAT), lambda i: (0, 0)),
            pl.BlockSpec((D, LAT), lambda i: (0, 0)),
            pl.BlockSpec((1, LAT), lambda i: (0, 0)),
            pl.BlockSpec((NB, LAT), lambda i: (clamp(i), 0)),
            pl.BlockSpec((1, N), lambda i: (0, 0)),
        ],
        out_specs=[
            pl.BlockSpec((NB, LAT), lambda i: (clamp(i), 0)),
            pl.BlockSpec((NB, LAT), lambda i: (clamp(i), 0)),
            pl.BlockSpec((NB, LAT), lambda i: (i, 0)),
            pl.BlockSpec((1, G), lambda i: (0, 0)),
            pl.BlockSpec((1, G), lambda i: (0, 0)),
        ],
        out_shape=[
            jax.ShapeDtypeStruct((N, LAT), jnp.float32),
            jax.ShapeDtypeStruct((N, LAT), jnp.float32),
            jax.ShapeDtypeStruct((ZP, LAT), jnp.float32),
            jax.ShapeDtypeStruct((1, G), jnp.int32),
            jax.ShapeDtypeStruct((1, G), jnp.int32),
        ],
    )(agg, hs, dinv, b2, wmu, bmu, wlv, blv, eps, batch2)


def _dec_body(offs_ref, cnts_ref, z_ref, bias_ref, adj_ref, mask_ref):
    g = pl.program_id(0)
    off = offs_ref[g]
    cnt = cnts_ref[g]
    zb = z_ref[pl.ds(off, MN), :]
    colmask = lax.broadcasted_iota(jnp.int32, (MN, 1), 0) < cnt
    zm = jnp.where(colmask, zb, 0.0)
    logits = lax.dot_general(zm, zm, (((1,), (1,)), ((), ())),
                             preferred_element_type=jnp.float32,
                             precision=lax.Precision.HIGHEST)
    logits = logits * (LAT ** -0.5) + bias_ref[0]
    adj_ref[0] = jax.nn.sigmoid(logits)
    rowmask = lax.broadcasted_iota(jnp.int32, (1, MN), 1) < cnt
    mask_ref[0] = rowmask.astype(jnp.int32)


def _tc_decoder(offs, cnts, zpad, bias1):
    grid_spec = pltpu.PrefetchScalarGridSpec(
        num_scalar_prefetch=2,
        grid=(G,),
        in_specs=[
            pl.BlockSpec((ZP, LAT), lambda g, o, c: (0, 0)),
            pl.BlockSpec((1,), lambda g, o, c: (0,)),
        ],
        out_specs=[
            pl.BlockSpec((1, MN, MN), lambda g, o, c: (g, 0, 0)),
            pl.BlockSpec((1, 1, MN), lambda g, o, c: (g, 0, 0)),
        ],
    )
    return pl.pallas_call(
        _dec_body,
        grid_spec=grid_spec,
        out_shape=[
            jax.ShapeDtypeStruct((G, MN, MN), jnp.float32),
            jax.ShapeDtypeStruct((G, 1, MN), jnp.int32),
        ],
    )(offs, cnts, zpad, bias1)


def kernel(x, edge_index, batch, eps, W1, b1, W2, b2, Wmu, bmu, Wlv, blv,
           dec_bias):
    src = edge_index[0]
    dst = edge_index[1]
    npad = IRX * 128 - E
    padi = jnp.arange(npad, dtype=jnp.int32)
    srcp = jnp.concatenate([src, (padi * 97) % N])
    dstp = jnp.concatenate([dst, N + padi % NTRASH])
    dst3 = dstp.reshape(IRX, 128)
    srcs3 = jnp.stack([srcp, srcp + N], axis=0).reshape(NC, IRX, 128)
    ones128 = jnp.ones((128,), jnp.float32)
    zeros1 = jnp.zeros((CPT + TAIL,), jnp.float32)
    zerosC = jnp.zeros((CC, HW), jnp.float32)

    deg2 = _sc_degree(dst3, ones128, zeros1)

    w1s = jnp.stack([W1[:, :HW], W1[:, HW:]], axis=0)
    h1 = _tc_mm1(x, w1s)
    hs1, dinv = _tc_scale1(deg2.reshape(NC, N, 1), h1)
    agg1 = _sc_aggregate(hs1.reshape(NC * N, HW), srcs3, dst3, zerosC)

    hs2 = _tc2(agg1, hs1, dinv, b1.reshape(1, D), W2)
    agg2 = _sc_aggregate(hs2.reshape(NC * N, HW), srcs3, dst3, zerosC)

    mu, logvar, zpad, offs, cnts = _tc3(
        agg2, hs2, dinv, b2.reshape(1, D), Wmu, bmu.reshape(1, LAT),
        Wlv, blv.reshape(1, LAT), eps, batch.reshape(1, N))

    adj, maski = _tc_decoder(offs.reshape(G), cnts.reshape(G), zpad,
                             dec_bias.reshape(1))
    return adj, mu, logvar, maski.reshape(G, MN).astype(bool)


# speedup vs baseline: 2.1356x; 1.0242x over previous
import functools

import jax
import jax.numpy as jnp
from jax import lax
from jax.experimental import pallas as pl
from jax.experimental.pallas import tpu as pltpu
from jax.experimental.pallas import tpu_sc as plsc

N = 10000
E = 320000
D = 128
HW = 64
LAT = 64
G = 16
MN = 1024
NC, NS = 2, 16

IROWS = E // 128
IPAD = 2560
IRX = IPAD + 8
NTRASH = 512
DRPW = IPAD // (NC * NS)
DCH = 8
ARPT = IPAD // NS
ACH = 4
PCH = ARPT // ACH
CPT = 624
TAIL = N - CPT * NS
CC = 104

_SC_PARAMS = pltpu.CompilerParams(use_tc_tiling_on_sc=False)


def _sc_mesh():
    return plsc.VectorSubcoreMesh(core_axis_name="c", subcore_axis_name="s")


def _deg_body(dst_hbm, ones_hbm, zeros_hbm, out_hbm, dbuf, ones_v, stage, acc,
              sem):
    c = lax.axis_index("c")
    s = lax.axis_index("s")
    pltpu.sync_copy(zeros_hbm, stage)
    pltpu.sync_copy(stage.at[pl.ds(0, CPT)], acc.at[pl.ds(s * CPT, CPT)])

    @pl.when(s == NS - 1)
    def _():
        pltpu.sync_copy(stage.at[pl.ds(0, TAIL)],
                        acc.at[pl.ds(CPT * NS, TAIL)])

    pltpu.sync_copy(ones_hbm, ones_v)
    base = (c * NS + s) * DRPW
    pltpu.sync_copy(dst_hbm.at[pl.ds(base, DRPW), :], dbuf)
    plsc.subcore_barrier()

    ndch = jnp.where(s >= 0, DRPW // DCH, 0)

    def chunk(i, carry):
        descs = [
            pltpu.async_copy(ones_v, acc.at[dbuf.at[i * DCH + j]], sem,
                             add=True)
            for j in range(DCH)
        ]
        for d in descs:
            d.wait()
        return carry

    lax.fori_loop(0, ndch, chunk, 0)
    plsc.subcore_barrier()
    pltpu.sync_copy(acc.at[pl.ds(s * CPT, CPT)], stage.at[pl.ds(0, CPT)])
    pltpu.sync_copy(stage.at[pl.ds(0, CPT)],
                    out_hbm.at[pl.ds(c * N + s * CPT, CPT)])

    @pl.when(s == NS - 1)
    def _():
        pltpu.sync_copy(acc.at[pl.ds(CPT * NS, TAIL)],
                        stage.at[pl.ds(CPT, TAIL)])
        pltpu.sync_copy(stage.at[pl.ds(CPT, TAIL)],
                        out_hbm.at[pl.ds(c * N + CPT * NS, TAIL)])


def _sc_degree(dst3, ones128, zeros1):
    return pl.kernel(
        _deg_body,
        out_type=jax.ShapeDtypeStruct((NC * N,), jnp.float32),
        mesh=_sc_mesh(),
        scratch_types=[
            pltpu.VMEM((DRPW, 128), jnp.int32),
            pltpu.VMEM((128,), jnp.float32),
            pltpu.VMEM((CPT + TAIL,), jnp.float32),
            pltpu.VMEM_SHARED((N + NTRASH,), jnp.float32),
            pltpu.SemaphoreType.DMA,
        ],
        compiler_params=_SC_PARAMS,
    )(dst3, ones128, zeros1)


def _agg_body(hs_hbm, srcs_hbm, dst_hbm, zeros_hbm, out_hbm,
              sbufA, dbufA, sbufB, dbufB, rowsA, rowsB, stage, acc,
              semA, semB, isemB):
    c = lax.axis_index("c")
    s = lax.axis_index("s")
    pltpu.sync_copy(zeros_hbm, stage)
    for k in range(CPT // CC):
        pltpu.sync_copy(stage, acc.at[pl.ds(s * CPT + k * CC, CC)])

    @pl.when(s == NS - 1)
    def _():
        pltpu.sync_copy(stage.at[pl.ds(0, TAIL), :],
                        acc.at[pl.ds(CPT * NS, TAIL)])

    base = s * ARPT
    plsc.subcore_barrier()

    def idx_load(sbuf, dbuf, ch):
        rb = base + ch * ACH
        pltpu.sync_copy(srcs_hbm.at[c, pl.ds(rb, ACH), :], sbuf)
        pltpu.sync_copy(dst_hbm.at[pl.ds(rb, ACH), :], dbuf)

    def fire(sbuf, rows, sem):
        for j in range(ACH):
            pltpu.async_copy(hs_hbm.at[sbuf.at[j]],
                             rows.at[pl.ds(j * 128, 128)], sem)

    def gather_wait(sbuf, rows, sem):
        for j in range(ACH):
            pltpu.make_async_copy(hs_hbm.at[sbuf.at[j]],
                                  rows.at[pl.ds(j * 128, 128)], sem).wait()

    def scatter(dbuf, rows):
        for j in range(ACH):
            pltpu.sync_copy(rows.at[pl.ds(j * 128, 128)],
                            acc.at[dbuf.at[j]], add=True)

    def idx_fire(sbuf, dbuf, isem, ch):
        rb = base + ch * ACH
        pltpu.async_copy(srcs_hbm.at[c, pl.ds(rb, ACH), :], sbuf, isem)
        pltpu.async_copy(dst_hbm.at[pl.ds(rb, ACH), :], dbuf, isem)

    def idx_wait(sbuf, dbuf, isem, ch):
        rb = base + ch * ACH
        pltpu.make_async_copy(srcs_hbm.at[c, pl.ds(rb, ACH), :], sbuf,
                              isem).wait()
        pltpu.make_async_copy(dst_hbm.at[pl.ds(rb, ACH), :], dbuf,
                              isem).wait()

    idx_load(sbufA, dbufA, 0)
    fire(sbufA, rowsA, semA)
    idx_fire(sbufB, dbufB, isemB, 1)
    npair = jnp.where(s >= 0, PCH // 2, 0)

    def body(p, carry):
        a = 2 * p
        idx_wait(sbufB, dbufB, isemB, a + 1)
        gather_wait(sbufA, rowsA, semA)
        fire(sbufB, rowsB, semB)
        scatter(dbufA, rowsA)
        idx_load(sbufA, dbufA, a + 2)
        gather_wait(sbufB, rowsB, semB)
        fire(sbufA, rowsA, semA)
        scatter(dbufB, rowsB)
        idx_fire(sbufB, dbufB, isemB, a + 3)
        return carry

    lax.fori_loop(0, npair, body, 0)
    gather_wait(sbufA, rowsA, semA)
    idx_wait(sbufB, dbufB, isemB, PCH + 1)
    plsc.subcore_barrier()
    for k in range(CPT // CC):
        pltpu.sync_copy(acc.at[pl.ds(s * CPT + k * CC, CC)], stage)
        pltpu.sync_copy(stage, out_hbm.at[c, pl.ds(s * CPT + k * CC, CC), :])

    @pl.when(s == NS - 1)
    def _():
        pltpu.sync_copy(acc.at[pl.ds(CPT * NS, TAIL)],
                        stage.at[pl.ds(0, TAIL), :])
        pltpu.sync_copy(stage.at[pl.ds(0, TAIL), :],
                        out_hbm.at[c, pl.ds(CPT * NS, TAIL), :])


def _sc_aggregate(hs_flat, srcs3, dst3, zerosC):
    return pl.kernel(
        _agg_body,
        out_type=jax.ShapeDtypeStruct((NC, N, HW), jnp.float32),
        mesh=_sc_mesh(),
        scratch_types=[
            pltpu.VMEM((ACH, 128), jnp.int32),
            pltpu.VMEM((ACH, 128), jnp.int32),
            pltpu.VMEM((ACH, 128), jnp.int32),
            pltpu.VMEM((ACH, 128), jnp.int32),
            pltpu.VMEM((ACH * 128, HW), jnp.float32),
            pltpu.VMEM((ACH * 128, HW), jnp.float32),
            pltpu.VMEM((CC, HW), jnp.float32),
            pltpu.VMEM_SHARED((N + NTRASH, HW), jnp.float32),
            pltpu.SemaphoreType.DMA,
            pltpu.SemaphoreType.DMA,
            pltpu.SemaphoreType.DMA,
        ],
        compiler_params=_SC_PARAMS,
    )(hs_flat, srcs3, dst3, zerosC)


def _mm1_body(x_ref, w_ref, h_ref):
    h_ref[0] = jnp.dot(x_ref[...], w_ref[0], preferred_element_type=jnp.float32,
                       precision=lax.Precision.HIGHEST)


def _tc_mm1(x, w1s):
    return pl.pallas_call(
        _mm1_body,
        grid=(2,),
        in_specs=[
            pl.BlockSpec((N, D), lambda f: (0, 0)),
            pl.BlockSpec((1, D, HW), lambda f: (f, 0, 0)),
        ],
        out_specs=pl.BlockSpec((1, N, HW), lambda f: (f, 0, 0)),
        out_shape=jax.ShapeDtypeStruct((NC, N, HW), jnp.float32),
    )(x, w1s)


def _scale1_body(deg_ref, h_ref, hs_ref, dinv_ref):
    degsum = deg_ref[0] + deg_ref[1] + 1.0
    dinv = lax.rsqrt(degsum)
    dinv_ref[...] = dinv
    hs_ref[0] = h_ref[0] * dinv
    hs_ref[1] = h_ref[1] * dinv


def _tc_scale1(deg2, h1):
    return pl.pallas_call(
        _scale1_body,
        out_shape=[
            jax.ShapeDtypeStruct((NC, N, HW), jnp.float32),
            jax.ShapeDtypeStruct((N, 1), jnp.float32),
        ],
    )(deg2, h1)


def _tc2_body(agg_ref, hs_ref, dinv_ref, b_ref, w_ref, out_ref):
    dinv = dinv_ref[...]
    h0 = dinv * (agg_ref[0] + hs_ref[0])
    h1 = dinv * (agg_ref[1] + hs_ref[1])
    h = jnp.concatenate([h0, h1], axis=1) + b_ref[...]
    h = jnp.maximum(h, 0.0)
    h2 = jnp.dot(h, w_ref[...], preferred_element_type=jnp.float32,
                 precision=lax.Precision.HIGHEST)
    out_ref[0] = dinv * h2[:, :HW]
    out_ref[1] = dinv * h2[:, HW:]


NB = 2000


def _tc2(agg, hs, dinv, b1, w2):
    return pl.pallas_call(
        _tc2_body,
        grid=(N // NB,),
        in_specs=[
            pl.BlockSpec((NC, NB, HW), lambda i: (0, i, 0)),
            pl.BlockSpec((NC, NB, HW), lambda i: (0, i, 0)),
            pl.BlockSpec((NB, 1), lambda i: (i, 0)),
            pl.BlockSpec((1, D), lambda i: (0, 0)),
            pl.BlockSpec((D, D), lambda i: (0, 0)),
        ],
        out_specs=pl.BlockSpec((NC, NB, HW), lambda i: (0, i, 0)),
        out_shape=jax.ShapeDtypeStruct((NC, N, HW), jnp.float32),
    )(agg, hs, dinv, b1, w2)


ZP = N + NB


def _tc3_body(agg_ref, hs_ref, dinv_ref, b_ref, wmu_ref, bmu_ref, wlv_ref,
              blv_ref, eps_ref, batch_ref,
              mu_ref, lv_ref, zpad_ref, offs_ref, cnts_ref):
    i = pl.program_id(0)
    dinv = dinv_ref[...]
    h0 = dinv * (agg_ref[0] + hs_ref[0])
    h1 = dinv * (agg_ref[1] + hs_ref[1])
    h = jnp.concatenate([h0, h1], axis=1) + b_ref[...]
    h = jnp.maximum(h, 0.0)
    mu = jnp.dot(h, wmu_ref[...], preferred_element_type=jnp.float32,
                 precision=lax.Precision.HIGHEST) + bmu_ref[...]
    logvar = jnp.dot(h, wlv_ref[...], preferred_element_type=jnp.float32,
                     precision=lax.Precision.HIGHEST) + blv_ref[...]
    mu_ref[...] = mu
    lv_ref[...] = logvar
    lvc = jnp.clip(logvar, -20.0, 20.0)
    z = mu + eps_ref[...] * jnp.exp(0.5 * lvc)
    zpad_ref[...] = jnp.where(i < N // NB, z, 0.0)
    gids = lax.broadcasted_iota(jnp.int32, (G, N), 0)
    b = batch_ref[...]
    cnts_ref[...] = jnp.sum((b == gids).astype(jnp.int32), axis=1).reshape(1, G)
    offs_ref[...] = jnp.sum((b < gids).astype(jnp.int32), axis=1).reshape(1, G)


def _tc3(agg, hs, dinv, b2, wmu, bmu, wlv, blv, eps, batch2):
    clamp = lambda i: jnp.minimum(i, N // NB - 1)
    return pl.pallas_call(
        _tc3_body,
        grid=(ZP // NB,),
        in_specs=[
            pl.BlockSpec((NC, NB, HW), lambda i: (0, clamp(i), 0)),
            pl.BlockSpec((NC, NB, HW), lambda i: (0, clamp(i), 0)),
            pl.BlockSpec((NB, 1), lambda i: (clamp(i), 0)),
            pl.BlockSpec((1, D), lambda i: (0, 0)),
            pl.BlockSpec((D, LAT), lambda i: (0, 0)),
            pl.BlockSpec((1, LAT), lambda i: (0, 0)),
            pl.BlockSpec((D, LAT), lambda i: (0, 0)),
            pl.BlockSpec((1, LAT), lambda i: (0, 0)),
            pl.BlockSpec((NB, LAT), lambda i: (clamp(i), 0)),
            pl.BlockSpec((1, N), lambda i: (0, 0)),
        ],
        out_specs=[
            pl.BlockSpec((NB, LAT), lambda i: (clamp(i), 0)),
            pl.BlockSpec((NB, LAT), lambda i: (clamp(i), 0)),
            pl.BlockSpec((NB, LAT), lambda i: (i, 0)),
            pl.BlockSpec((1, G), lambda i: (0, 0)),
            pl.BlockSpec((1, G), lambda i: (0, 0)),
        ],
        out_shape=[
            jax.ShapeDtypeStruct((N, LAT), jnp.float32),
            jax.ShapeDtypeStruct((N, LAT), jnp.float32),
            jax.ShapeDtypeStruct((ZP, LAT), jnp.float32),
            jax.ShapeDtypeStruct((1, G), jnp.int32),
            jax.ShapeDtypeStruct((1, G), jnp.int32),
        ],
    )(agg, hs, dinv, b2, wmu, bmu, wlv, blv, eps, batch2)


def _dec_body(offs_ref, cnts_ref, z_ref, bias_ref, adj_ref, mask_ref):
    g = pl.program_id(0)
    off = offs_ref[g]
    cnt = cnts_ref[g]
    zb = z_ref[pl.ds(off, MN), :]
    colmask = lax.broadcasted_iota(jnp.int32, (MN, 1), 0) < cnt
    zm = jnp.where(colmask, zb, 0.0)
    logits = lax.dot_general(zm, zm, (((1,), (1,)), ((), ())),
                             preferred_element_type=jnp.float32,
                             precision=lax.Precision.HIGHEST)
    logits = logits * (LAT ** -0.5) + bias_ref[0]
    adj_ref[0] = jax.nn.sigmoid(logits)
    rowmask = lax.broadcasted_iota(jnp.int32, (1, MN), 1) < cnt
    mask_ref[0] = rowmask.astype(jnp.int32)


def _tc_decoder(offs, cnts, zpad, bias1):
    grid_spec = pltpu.PrefetchScalarGridSpec(
        num_scalar_prefetch=2,
        grid=(G,),
        in_specs=[
            pl.BlockSpec((ZP, LAT), lambda g, o, c: (0, 0)),
            pl.BlockSpec((1,), lambda g, o, c: (0,)),
        ],
        out_specs=[
            pl.BlockSpec((1, MN, MN), lambda g, o, c: (g, 0, 0)),
            pl.BlockSpec((1, 1, MN), lambda g, o, c: (g, 0, 0)),
        ],
    )
    return pl.pallas_call(
        _dec_body,
        grid_spec=grid_spec,
        out_shape=[
            jax.ShapeDtypeStruct((G, MN, MN), jnp.float32),
            jax.ShapeDtypeStruct((G, 1, MN), jnp.int32),
        ],
    )(offs, cnts, zpad, bias1)


def kernel(x, edge_index, batch, eps, W1, b1, W2, b2, Wmu, bmu, Wlv, blv,
           dec_bias):
    src = edge_index[0]
    dst = edge_index[1]
    npad = IRX * 128 - E
    padi = jnp.arange(npad, dtype=jnp.int32)
    srcp = jnp.concatenate([src, (padi * 97) % N])
    dstp = jnp.concatenate([dst, N + padi % NTRASH])
    dst3 = dstp.reshape(IRX, 128)
    srcs3 = jnp.stack([srcp, srcp + N], axis=0).reshape(NC, IRX, 128)
    ones128 = jnp.ones((128,), jnp.float32)
    zeros1 = jnp.zeros((CPT + TAIL,), jnp.float32)
    zerosC = jnp.zeros((CC, HW), jnp.float32)

    deg2 = _sc_degree(dst3, ones128, zeros1)

    w1s = jnp.stack([W1[:, :HW], W1[:, HW:]], axis=0)
    h1 = _tc_mm1(x, w1s)
    hs1, dinv = _tc_scale1(deg2.reshape(NC, N, 1), h1)
    agg1 = _sc_aggregate(hs1.reshape(NC * N, HW), srcs3, dst3, zerosC)

    hs2 = _tc2(agg1, hs1, dinv, b1.reshape(1, D), W2)
    agg2 = _sc_aggregate(hs2.reshape(NC * N, HW), srcs3, dst3, zerosC)

    mu, logvar, zpad, offs, cnts = _tc3(
        agg2, hs2, dinv, b2.reshape(1, D), Wmu, bmu.reshape(1, LAT),
        Wlv, blv.reshape(1, LAT), eps, batch.reshape(1, N))

    adj, maski = _tc_decoder(offs.reshape(G), cnts.reshape(G), zpad,
                             dec_bias.reshape(1))
    return adj, mu, logvar, maski.reshape(G, MN).astype(bool)
